# baseline (speedup 1.0000x reference)
import functools

import jax
import jax.numpy as jnp
from jax import lax
from jax.experimental import pallas as pl
from jax.experimental.pallas import tpu as pltpu
from jax.experimental.pallas import tpu_sc as plsc

_NPAD = 10240


def _edge_proj_body(*refs):
    nout = (len(refs) - 5) // 3
    attr_ref, w1_ref, b1_ref, w2_ref, b2_ref = refs[:5]
    wh_refs = refs[5:5 + nout]
    bh_refs = refs[5 + nout:5 + 2 * nout]
    out_refs = refs[5 + 2 * nout:]
    h = jnp.maximum(
        jnp.dot(attr_ref[...], w1_ref[...], preferred_element_type=jnp.float32)
        + b1_ref[...][None, :],
        0.0,
    )
    ea = (
        jnp.dot(h, w2_ref[...], preferred_element_type=jnp.float32)
        + b2_ref[...][None, :]
    )
    for wh_ref, bh_ref, out_ref in zip(wh_refs, bh_refs, out_refs):
        for c in range(2):
            out_ref[c, :, :] = (
                jnp.dot(ea, wh_ref[c], preferred_element_type=jnp.float32)
                + bh_ref[c][None, :]
            )


def _edge_proj(edge_attr, em, whs, bhs, block=3200):
    e, de = edge_attr.shape
    h = em[0]["w"].shape[1]
    w3 = whs[0].shape[2]
    nout = len(whs)
    return pl.pallas_call(
        _edge_proj_body,
        grid=(e // block,),
        in_specs=[
            pl.BlockSpec((block, de), lambda i: (i, 0)),
            pl.BlockSpec((de, h), lambda i: (0, 0)),
            pl.BlockSpec((h,), lambda i: (0,)),
            pl.BlockSpec((h, h), lambda i: (0, 0)),
            pl.BlockSpec((h,), lambda i: (0,)),
        ]
        + [pl.BlockSpec((2, h, w3), lambda i: (0, 0, 0))] * nout
        + [pl.BlockSpec((2, w3), lambda i: (0, 0))] * nout,
        out_specs=[pl.BlockSpec((2, block, w3), lambda i: (0, i, 0))] * nout,
        out_shape=[jax.ShapeDtypeStruct((2, e, w3), jnp.float32)] * nout,
    )(edge_attr, em[0]["w"], em[0]["b"], em[1]["w"], em[1]["b"], *whs, *bhs)


@functools.cache
def _make_sc_agg(n, e, w):
    assert w == 32
    w3 = 3 * w
    B = 80
    TILES = 16
    DEPTH = 3
    epw = e // TILES
    nchunks = epw // B
    npt = n // TILES
    ZR = 64
    assert epw * TILES == e and nchunks * B == epw
    assert nchunks >= 4 * DEPTH
    assert npt * TILES == n and npt % ZR == 0 and npt % 8 == 0
    ngroups = (nchunks - 2 * DEPTH - 1) // DEPTH
    lo_tail = DEPTH + ngroups * DEPTH
    tail = list(range(lo_tail, nchunks))

    mesh = plsc.VectorSubcoreMesh(core_axis_name="c", subcore_axis_name="s")

    @functools.partial(
        pl.kernel,
        out_type=jax.ShapeDtypeStruct((2 * n, w3), jnp.float32),
        mesh=mesh,
        compiler_params=pltpu.CompilerParams(use_tc_tiling_on_sc=False),
        scratch_types=[
            pltpu.VMEM((2, B), jnp.int32),
            pltpu.VMEM((2, B), jnp.int32),
            pltpu.VMEM((2, B), jnp.int32),
            pltpu.VMEM((B,), jnp.int32),
            pltpu.VMEM((B,), jnp.int32),
            pltpu.VMEM((B,), jnp.int32),
            pltpu.VMEM((B,), jnp.int32),
            pltpu.VMEM((B,), jnp.int32),
            pltpu.VMEM((B,), jnp.int32),
            pltpu.VMEM((B, w), jnp.float32),
            pltpu.VMEM((B, w), jnp.float32),
            pltpu.VMEM((B, w), jnp.float32),
            pltpu.VMEM((B, w3), jnp.float32),
            pltpu.VMEM((B, w3), jnp.float32),
            pltpu.VMEM((B, w3), jnp.float32),
            pltpu.VMEM((B, w3), jnp.float32),
            pltpu.VMEM((B, w3), jnp.float32),
            pltpu.VMEM((B, w3), jnp.float32),
            pltpu.VMEM((ZR, w3), jnp.float32),
            pltpu.VMEM_SHARED((n, w3), jnp.float32),
        ] + [pltpu.SemaphoreType.DMA] * (4 * DEPTH),
    )
    def sc_agg(xh, eh, sd, out, sv0, sv1, sv2, dv0, dv1, dv2,
               iv0, iv1, iv2, xv0, xv1, xv2, ev0, ev1, ev2, mv0, mv1, mv2,
               zbuf, agg_sh, *sems):
        c = lax.axis_index("c")
        s = lax.axis_index("s")
        cn = c * n
        sdv, dst_v, idx_v = (sv0, sv1, sv2), (dv0, dv1, dv2), (iv0, iv1, iv2)
        xs_v, e_v, m_v = (xv0, xv1, xv2), (ev0, ev1, ev2), (mv0, mv1, mv2)
        s_sd = sems[0:3]
        s_g = sems[3:6]
        s_e = sems[6:9]
        s_s = sems[9:12]

        def sd_sl(k):
            return sd.at[s * nchunks + k]

        def eh_sl(k):
            return eh.at[pl.ds(c * e + s * epw + k * B, B)]

        def zrow(r, carry):
            for g in range(w3 // 16):
                zbuf[r, pl.ds(g * 16, 16)] = jnp.zeros((16,), jnp.float32)
            return carry

        lax.fori_loop(0, ZR, zrow, 0)
        row0 = s * npt
        for j in range(npt // ZR):
            pltpu.sync_copy(zbuf, agg_sh.at[pl.ds(row0 + j * ZR, ZR)])
        plsc.subcore_barrier()

        def start_gather(k, p):
            for g in range(B // 16):
                idx_v[p][pl.ds(g * 16, 16)] = (
                    sdv[p][0, pl.ds(g * 16, 16)] + cn
                )
            pltpu.async_copy(xh.at[idx_v[p]], xs_v[p], s_g[p])
            pltpu.async_copy(eh_sl(k), e_v[p], s_e[p])

        def compute(p):
            UNROLL = 4

            def edge_row(b4, carry2):
                for u in range(UNROLL):
                    b = UNROLL * b4 + u
                    xa = xs_v[p][b, pl.ds(0, 16)]
                    xb = xs_v[p][b, pl.ds(16, 16)]
                    for i in range(3):
                        m_v[p][b, pl.ds(i * 32, 16)] = jnp.maximum(
                            xa + e_v[p][b, pl.ds(i * 16, 16)], 0.0)
                        m_v[p][b, pl.ds(i * 32 + 16, 16)] = jnp.maximum(
                            xb + e_v[p][b, pl.ds(48 + i * 16, 16)], 0.0)
                return carry2

            lax.fori_loop(0, B // UNROLL, edge_row, 0)

        def wait_scatter(p):
            pltpu.make_async_copy(m_v[p], agg_sh.at[pl.ds(0, B)],
                                  s_s[p]).wait()

        def body(k, slot, *, boot=False, pre_gather=True, pre_src=True):
            g_slot = (slot + 2) % DEPTH
            if pre_gather:
                pltpu.make_async_copy(sd_sl(0), sdv[g_slot],
                                      s_sd[g_slot]).wait()
                start_gather(k + 2, g_slot)
            if not boot:
                wait_scatter(slot)
            for g in range(B // 16):
                dst_v[slot][pl.ds(g * 16, 16)] = sdv[slot][1, pl.ds(g * 16, 16)]
            pltpu.make_async_copy(xh.at[pl.ds(0, B)], xs_v[slot],
                                  s_g[slot]).wait()
            pltpu.make_async_copy(eh_sl(0), e_v[slot], s_e[slot]).wait()
            compute(slot)
            pltpu.async_copy(m_v[slot], agg_sh.at[dst_v[slot]], s_s[slot],
                             add=True)
            if pre_src:
                pltpu.async_copy(sd_sl(k + DEPTH), sdv[slot], s_sd[slot])

        for p in range(DEPTH):
            pltpu.async_copy(sd_sl(p), sdv[p], s_sd[p])
        for p in range(2):
            pltpu.make_async_copy(sd_sl(0), sdv[p], s_sd[p]).wait()
            start_gather(p, p)
        for k in range(DEPTH):
            body(k, k, boot=True)

        def group(g, carry):
            k0 = DEPTH * g
            for j in range(DEPTH):
                body(k0 + j, j)
            return carry

        lax.fori_loop(1, 1 + ngroups, group, 0)
        for k in tail:
            body(k, k % DEPTH,
                 pre_gather=(k + 2 < nchunks),
                 pre_src=(k + DEPTH < nchunks))
        for p in range(DEPTH):
            wait_scatter(p)
        plsc.subcore_barrier()

        pltpu.sync_copy(agg_sh.at[pl.ds(row0, npt)],
                        out.at[pl.ds(cn + row0, npt)])

    return sc_agg


def _node_layer_body(x_refs, agg_refs, w1s, b1s, w2s, b2s, wlo, blo):
    x = (x_refs[0][...] if len(x_refs[0].shape) == 2 else
         jnp.concatenate([r[c, :, :] for r in x_refs for c in range(2)],
                         axis=1))
    d = x.shape[1]
    outs = []
    for i in range(3):
        agg_i = jnp.concatenate(
            [a[c, :, i * 32:(i + 1) * 32] for a in agg_refs for c in range(2)],
            axis=1,
        )
        zi = x + agg_i
        t = jnp.maximum(
            jnp.dot(zi, w1s[i], preferred_element_type=jnp.float32)
            + b1s[i][None, :], 0.0)
        outs.append(
            jnp.dot(t, w2s[i], preferred_element_type=jnp.float32)
            + b2s[i][None, :])
    h = jnp.concatenate(outs, axis=1)
    return jnp.maximum(
        jnp.dot(h, wlo[...], preferred_element_type=jnp.float32)
        + blo[...][None, :], 0.0)


def _node_layer1_kernel(x_ref, agg_a, agg_b, w1s, b1s, w2s, b2s, wlo, blo,
                        xq_ref):
    xn = _node_layer_body((x_ref,), (agg_a, agg_b), w1s, b1s, w2s, b2s,
                          wlo, blo)
    xq_ref[0, :, :] = xn[:, :32]
    xq_ref[1, :, :] = xn[:, 32:]


def _node_layer2_kernel(xq1_ref, agg_a, w1s, b1s, w2s, b2s, wlo, blo,
                        sum_ref):
    xn = _node_layer_body((xq1_ref,), (agg_a,), w1s, b1s, w2s, b2s, wlo, blo)

    @pl.when(pl.program_id(0) == 0)
    def _():
        sum_ref[...] = jnp.zeros_like(sum_ref)

    sum_ref[...] += jnp.sum(xn, axis=0, keepdims=True)


def _stack_nn(convs):
    w1s = jnp.stack([convs[i]["nn"][0]["w"] for i in range(3)])
    b1s = jnp.stack([convs[i]["nn"][0]["b"] for i in range(3)])
    w2s = jnp.stack([convs[i]["nn"][1]["w"] for i in range(3)])
    b2s = jnp.stack([convs[i]["nn"][1]["b"] for i in range(3)])
    return w1s, b1s, w2s, b2s


def _node_layer(x_or_xq, aggs, convs, lin_out, last, blk=2000):
    w1s, b1s, w2s, b2s = _stack_nn(convs)
    n = 10000
    d = w1s.shape[1]
    grid = (n // blk,)
    wspecs = [
        pl.BlockSpec(w1s.shape, lambda i: (0,) * 3),
        pl.BlockSpec(b1s.shape, lambda i: (0,) * 2),
        pl.BlockSpec(w2s.shape, lambda i: (0,) * 3),
        pl.BlockSpec(b2s.shape, lambda i: (0,) * 2),
        pl.BlockSpec(lin_out["w"].shape, lambda i: (0,) * 2),
        pl.BlockSpec(lin_out["b"].shape, lambda i: (0,)),
    ]
    agg_specs = [pl.BlockSpec((2, blk, 96), lambda i: (0, i, 0))] * len(aggs)
    if not last:
        return pl.pallas_call(
            _node_layer1_kernel,
            grid=grid,
            in_specs=[pl.BlockSpec((blk, d), lambda i: (i, 0))]
            + agg_specs + wspecs,
            out_specs=pl.BlockSpec((2, blk, 32), lambda i: (0, i, 0)),
            out_shape=jax.ShapeDtypeStruct((2, _NPAD, 32), jnp.float32),
        )(x_or_xq, *aggs, w1s, b1s, w2s, b2s, lin_out["w"], lin_out["b"])
    return pl.pallas_call(
        _node_layer2_kernel,
        grid=grid,
        in_specs=[pl.BlockSpec((2, blk, 32), lambda i: (0, i, 0))]
        + agg_specs + wspecs,
        out_specs=pl.BlockSpec((1, d), lambda i: (0, 0)),
        out_shape=jax.ShapeDtypeStruct((1, d), jnp.float32),
    )(x_or_xq, *aggs, w1s, b1s, w2s, b2s, lin_out["w"], lin_out["b"])


def _apply_lin(p, x):
    return x @ p["w"] + p["b"]


def _mlp(ps, x):
    return _apply_lin(ps[1], jnp.maximum(_apply_lin(ps[0], x), 0.0))


def _col_group_weights(convs, q, w):
    hw = w // 2
    cols = [convs[i]["lin"]["w"][:, q * w + h * hw:q * w + (h + 1) * hw]
            for h in range(2) for i in range(3)]
    bs = [convs[i]["lin"]["b"][q * w + h * hw:q * w + (h + 1) * hw]
          for h in range(2) for i in range(3)]
    return jnp.concatenate(cols, axis=1), jnp.concatenate(bs)


def _gine_aggs(xhs, src_dst_packed, edge_attr, em, convs):
    e = src_dst_packed.shape[0] * src_dst_packed.shape[2]
    w = 32
    ncall = len(xhs)
    npad = _NPAD
    whs, bhs = [], []
    for r in range(ncall):
        pair = [_col_group_weights(convs, 2 * r + c, w) for c in range(2)]
        whs.append(jnp.stack([p[0] for p in pair]))
        bhs.append(jnp.stack([p[1] for p in pair]))
    ehs = _edge_proj(edge_attr, em, whs, bhs)
    if not isinstance(ehs, (list, tuple)):
        ehs = [ehs]
    sc = _make_sc_agg(npad, e, w)
    sd = src_dst_packed
    return [
        sc(xhs[r], ehs[r].reshape(2 * e, 3 * w), sd).reshape(
            2, npad, 3 * w)
        for r in range(ncall)
    ]


def kernel(x, edge_index, edge_attr, u, params):
    src = edge_index[0]
    dst = edge_index[1]
    n, d = x.shape
    npad = _NPAD
    xp = jnp.pad(x, ((0, npad - n), (0, 0)))
    xq = xp.reshape(npad, 4, 32).transpose(1, 0, 2)
    xhs1 = [xq[2 * r:2 * r + 2].reshape(2 * npad, 32) for r in range(2)]
    sd = edge_index.reshape(2, -1, 80).transpose(1, 0, 2)
    aggs1 = _gine_aggs(xhs1, sd, edge_attr, params["em1"], params["c1"])
    xq1 = _node_layer(x, aggs1, params["c1"], params["lin1"], last=False)
    aggs2 = _gine_aggs([xq1.reshape(2 * npad, 32)], sd, edge_attr,
                       params["em2"], params["c2"])
    sum2 = _node_layer(xq1, aggs2, params["c2"], params["lin2"], last=True)
    pooled = sum2 / n
    return _apply_lin(params["fc"], jnp.concatenate([pooled, u], axis=1))


# speedup vs baseline: 1.0136x; 1.0136x over previous
import functools

import jax
import jax.numpy as jnp
from jax import lax
from jax.experimental import pallas as pl
from jax.experimental.pallas import tpu as pltpu
from jax.experimental.pallas import tpu_sc as plsc

_NPAD = 10240


def _edge_proj_body(*refs):
    nout = (len(refs) - 5) // 3
    attr_ref, w1_ref, b1_ref, w2_ref, b2_ref = refs[:5]
    wh_refs = refs[5:5 + nout]
    bh_refs = refs[5 + nout:5 + 2 * nout]
    out_refs = refs[5 + 2 * nout:]
    h = jnp.maximum(
        jnp.dot(attr_ref[...], w1_ref[...], preferred_element_type=jnp.float32)
        + b1_ref[...][None, :],
        0.0,
    )
    ea = (
        jnp.dot(h, w2_ref[...], preferred_element_type=jnp.float32)
        + b2_ref[...][None, :]
    )
    for wh_ref, bh_ref, out_ref in zip(wh_refs, bh_refs, out_refs):
        for c in range(2):
            out_ref[c, :, :] = (
                jnp.dot(ea, wh_ref[c], preferred_element_type=jnp.float32)
                + bh_ref[c][None, :]
            )


def _edge_proj(edge_attr, em, whs, bhs, block=3200):
    e, de = edge_attr.shape
    h = em[0]["w"].shape[1]
    w3 = whs[0].shape[2]
    nout = len(whs)
    return pl.pallas_call(
        _edge_proj_body,
        grid=(e // block,),
        in_specs=[
            pl.BlockSpec((block, de), lambda i: (i, 0)),
            pl.BlockSpec((de, h), lambda i: (0, 0)),
            pl.BlockSpec((h,), lambda i: (0,)),
            pl.BlockSpec((h, h), lambda i: (0, 0)),
            pl.BlockSpec((h,), lambda i: (0,)),
        ]
        + [pl.BlockSpec((2, h, w3), lambda i: (0, 0, 0))] * nout
        + [pl.BlockSpec((2, w3), lambda i: (0, 0))] * nout,
        out_specs=[pl.BlockSpec((2, block, w3), lambda i: (0, i, 0))] * nout,
        out_shape=[jax.ShapeDtypeStruct((2, e, w3), jnp.float32)] * nout,
    )(edge_attr, em[0]["w"], em[0]["b"], em[1]["w"], em[1]["b"], *whs, *bhs)


@functools.cache
def _make_sc_agg(n, e, w):
    assert w == 32
    w3 = 3 * w
    B = 80
    TILES = 16
    DEPTH = 3
    epw = e // TILES
    nchunks = epw // B
    npt = n // TILES
    ZR = 64
    assert epw * TILES == e and nchunks * B == epw
    assert nchunks >= 4 * DEPTH
    assert npt * TILES == n and npt % ZR == 0 and npt % 8 == 0
    ngroups = (nchunks - 2 * DEPTH - 1) // DEPTH
    lo_tail = DEPTH + ngroups * DEPTH
    tail = list(range(lo_tail, nchunks))

    mesh = plsc.VectorSubcoreMesh(core_axis_name="c", subcore_axis_name="s")

    @functools.partial(
        pl.kernel,
        out_type=jax.ShapeDtypeStruct((2 * n, w3), jnp.float32),
        mesh=mesh,
        compiler_params=pltpu.CompilerParams(use_tc_tiling_on_sc=False),
        scratch_types=[
            pltpu.VMEM((B,), jnp.int32),
            pltpu.VMEM((B,), jnp.int32),
            pltpu.VMEM((B,), jnp.int32),
            pltpu.VMEM((B,), jnp.int32),
            pltpu.VMEM((B,), jnp.int32),
            pltpu.VMEM((B,), jnp.int32),
            pltpu.VMEM((B,), jnp.int32),
            pltpu.VMEM((B,), jnp.int32),
            pltpu.VMEM((B,), jnp.int32),
            pltpu.VMEM((B, w), jnp.float32),
            pltpu.VMEM((B, w), jnp.float32),
            pltpu.VMEM((B, w), jnp.float32),
            pltpu.VMEM((B, w3), jnp.float32),
            pltpu.VMEM((B, w3), jnp.float32),
            pltpu.VMEM((B, w3), jnp.float32),
            pltpu.VMEM((B, w3), jnp.float32),
            pltpu.VMEM((B, w3), jnp.float32),
            pltpu.VMEM((B, w3), jnp.float32),
            pltpu.VMEM((ZR, w3), jnp.float32),
            pltpu.VMEM_SHARED((n, w3), jnp.float32),
        ] + [pltpu.SemaphoreType.DMA] * (5 * DEPTH),
    )
    def sc_agg(xh, eh, src, dst, out, sv0, sv1, sv2, dv0, dv1, dv2,
               iv0, iv1, iv2, xv0, xv1, xv2, ev0, ev1, ev2, mv0, mv1, mv2,
               zbuf, agg_sh, *sems):
        c = lax.axis_index("c")
        s = lax.axis_index("s")
        cn = c * n
        src_v, dst_v, idx_v = (sv0, sv1, sv2), (dv0, dv1, dv2), (iv0, iv1, iv2)
        xs_v, e_v, m_v = (xv0, xv1, xv2), (ev0, ev1, ev2), (mv0, mv1, mv2)
        s_src = sems[0:3]
        s_dst = sems[3:6]
        s_g = sems[6:9]
        s_e = sems[9:12]
        s_s = sems[12:15]

        def src_sl(k):
            return src.at[pl.ds(s * epw + k * B, B)]

        def dst_sl(k):
            return dst.at[pl.ds(s * epw + k * B, B)]

        def eh_sl(k):
            return eh.at[pl.ds(c * e + s * epw + k * B, B)]

        def zrow(r, carry):
            for g in range(w3 // 16):
                zbuf[r, pl.ds(g * 16, 16)] = jnp.zeros((16,), jnp.float32)
            return carry

        lax.fori_loop(0, ZR, zrow, 0)
        row0 = s * npt
        for j in range(npt // ZR):
            pltpu.sync_copy(zbuf, agg_sh.at[pl.ds(row0 + j * ZR, ZR)])
        plsc.subcore_barrier()

        def start_gather(k, p):
            for g in range(B // 16):
                idx_v[p][pl.ds(g * 16, 16)] = (
                    src_v[p][pl.ds(g * 16, 16)] + cn
                )
            pltpu.async_copy(xh.at[idx_v[p]], xs_v[p], s_g[p])
            pltpu.async_copy(eh_sl(k), e_v[p], s_e[p])

        def compute(p):
            UNROLL = 4

            def edge_row(b4, carry2):
                for u in range(UNROLL):
                    b = UNROLL * b4 + u
                    xa = xs_v[p][b, pl.ds(0, 16)]
                    xb = xs_v[p][b, pl.ds(16, 16)]
                    for i in range(3):
                        m_v[p][b, pl.ds(i * 32, 16)] = jnp.maximum(
                            xa + e_v[p][b, pl.ds(i * 16, 16)], 0.0)
                        m_v[p][b, pl.ds(i * 32 + 16, 16)] = jnp.maximum(
                            xb + e_v[p][b, pl.ds(48 + i * 16, 16)], 0.0)
                return carry2

            lax.fori_loop(0, B // UNROLL, edge_row, 0)

        def wait_scatter(p):
            pltpu.make_async_copy(m_v[p], agg_sh.at[pl.ds(0, B)],
                                  s_s[p]).wait()

        def body(k, slot, *, boot=False, pre_gather=True, pre_src=True):
            g_slot = (slot + 2) % DEPTH
            if pre_gather:
                pltpu.make_async_copy(src_sl(0), src_v[g_slot],
                                      s_src[g_slot]).wait()
                start_gather(k + 2, g_slot)
            if not boot:
                wait_scatter(slot)
                pltpu.async_copy(dst_sl(k), dst_v[slot], s_dst[slot])
            pltpu.make_async_copy(xh.at[pl.ds(0, B)], xs_v[slot],
                                  s_g[slot]).wait()
            pltpu.make_async_copy(eh_sl(0), e_v[slot], s_e[slot]).wait()
            compute(slot)
            pltpu.make_async_copy(dst_sl(0), dst_v[slot], s_dst[slot]).wait()
            pltpu.async_copy(m_v[slot], agg_sh.at[dst_v[slot]], s_s[slot],
                             add=True)
            if pre_src:
                pltpu.async_copy(src_sl(k + DEPTH), src_v[slot], s_src[slot])

        for p in range(DEPTH):
            pltpu.async_copy(src_sl(p), src_v[p], s_src[p])
            pltpu.async_copy(dst_sl(p), dst_v[p], s_dst[p])
        for p in range(2):
            pltpu.make_async_copy(src_sl(0), src_v[p], s_src[p]).wait()
            start_gather(p, p)
        for k in range(DEPTH):
            body(k, k, boot=True)

        def group(g, carry):
            k0 = DEPTH * g
            for j in range(DEPTH):
                body(k0 + j, j)
            return carry

        lax.fori_loop(1, 1 + ngroups, group, 0)
        for k in tail:
            body(k, k % DEPTH,
                 pre_gather=(k + 2 < nchunks),
                 pre_src=(k + DEPTH < nchunks))
        for p in range(DEPTH):
            wait_scatter(p)
        plsc.subcore_barrier()

        pltpu.sync_copy(agg_sh.at[pl.ds(row0, npt)],
                        out.at[pl.ds(cn + row0, npt)])

    return sc_agg


def _node_layer_body(x_refs, agg_refs, w1s, b1s, w2s, b2s, wlo, blo):
    x = (x_refs[0][...] if len(x_refs[0].shape) == 2 else
         jnp.concatenate([r[c, :, :] for r in x_refs for c in range(2)],
                         axis=1))
    d = x.shape[1]
    outs = []
    for i in range(3):
        agg_i = jnp.concatenate(
            [a[c, :, i * 32:(i + 1) * 32] for a in agg_refs for c in range(2)],
            axis=1,
        )
        zi = x + agg_i
        t = jnp.maximum(
            jnp.dot(zi, w1s[i], preferred_element_type=jnp.float32)
            + b1s[i][None, :], 0.0)
        outs.append(
            jnp.dot(t, w2s[i], preferred_element_type=jnp.float32)
            + b2s[i][None, :])
    h = jnp.concatenate(outs, axis=1)
    return jnp.maximum(
        jnp.dot(h, wlo[...], preferred_element_type=jnp.float32)
        + blo[...][None, :], 0.0)


def _node_layer1_kernel(x_ref, agg_a, agg_b, w1s, b1s, w2s, b2s, wlo, blo,
                        xq_ref):
    xn = _node_layer_body((x_ref,), (agg_a, agg_b), w1s, b1s, w2s, b2s,
                          wlo, blo)
    xq_ref[0, :, :] = xn[:, :32]
    xq_ref[1, :, :] = xn[:, 32:]


def _node_layer2_kernel(xq1_ref, agg_a, w1s, b1s, w2s, b2s, wlo, blo,
                        sum_ref):
    xn = _node_layer_body((xq1_ref,), (agg_a,), w1s, b1s, w2s, b2s, wlo, blo)

    @pl.when(pl.program_id(0) == 0)
    def _():
        sum_ref[...] = jnp.zeros_like(sum_ref)

    sum_ref[...] += jnp.sum(xn, axis=0, keepdims=True)


def _stack_nn(convs):
    w1s = jnp.stack([convs[i]["nn"][0]["w"] for i in range(3)])
    b1s = jnp.stack([convs[i]["nn"][0]["b"] for i in range(3)])
    w2s = jnp.stack([convs[i]["nn"][1]["w"] for i in range(3)])
    b2s = jnp.stack([convs[i]["nn"][1]["b"] for i in range(3)])
    return w1s, b1s, w2s, b2s


def _node_layer(x_or_xq, aggs, convs, lin_out, last, blk=2000):
    w1s, b1s, w2s, b2s = _stack_nn(convs)
    n = 10000
    d = w1s.shape[1]
    grid = (n // blk,)
    wspecs = [
        pl.BlockSpec(w1s.shape, lambda i: (0,) * 3),
        pl.BlockSpec(b1s.shape, lambda i: (0,) * 2),
        pl.BlockSpec(w2s.shape, lambda i: (0,) * 3),
        pl.BlockSpec(b2s.shape, lambda i: (0,) * 2),
        pl.BlockSpec(lin_out["w"].shape, lambda i: (0,) * 2),
        pl.BlockSpec(lin_out["b"].shape, lambda i: (0,)),
    ]
    agg_specs = [pl.BlockSpec((2, blk, 96), lambda i: (0, i, 0))] * len(aggs)
    if not last:
        return pl.pallas_call(
            _node_layer1_kernel,
            grid=grid,
            in_specs=[pl.BlockSpec((blk, d), lambda i: (i, 0))]
            + agg_specs + wspecs,
            out_specs=pl.BlockSpec((2, blk, 32), lambda i: (0, i, 0)),
            out_shape=jax.ShapeDtypeStruct((2, _NPAD, 32), jnp.float32),
        )(x_or_xq, *aggs, w1s, b1s, w2s, b2s, lin_out["w"], lin_out["b"])
    return pl.pallas_call(
        _node_layer2_kernel,
        grid=grid,
        in_specs=[pl.BlockSpec((2, blk, 32), lambda i: (0, i, 0))]
        + agg_specs + wspecs,
        out_specs=pl.BlockSpec((1, d), lambda i: (0, 0)),
        out_shape=jax.ShapeDtypeStruct((1, d), jnp.float32),
    )(x_or_xq, *aggs, w1s, b1s, w2s, b2s, lin_out["w"], lin_out["b"])


def _apply_lin(p, x):
    return x @ p["w"] + p["b"]


def _mlp(ps, x):
    return _apply_lin(ps[1], jnp.maximum(_apply_lin(ps[0], x), 0.0))


def _col_group_weights(convs, q, w):
    hw = w // 2
    cols = [convs[i]["lin"]["w"][:, q * w + h * hw:q * w + (h + 1) * hw]
            for h in range(2) for i in range(3)]
    bs = [convs[i]["lin"]["b"][q * w + h * hw:q * w + (h + 1) * hw]
          for h in range(2) for i in range(3)]
    return jnp.concatenate(cols, axis=1), jnp.concatenate(bs)


def _gine_aggs(xhs, src, dst, edge_attr, em, convs):
    e = src.shape[0]
    w = 32
    ncall = len(xhs)
    npad = _NPAD
    whs, bhs = [], []
    for r in range(ncall):
        pair = [_col_group_weights(convs, 2 * r + c, w) for c in range(2)]
        whs.append(jnp.stack([p[0] for p in pair]))
        bhs.append(jnp.stack([p[1] for p in pair]))
    ehs = _edge_proj(edge_attr, em, whs, bhs)
    if not isinstance(ehs, (list, tuple)):
        ehs = [ehs]
    sc = _make_sc_agg(npad, e, w)
    return [
        sc(xhs[r], ehs[r].reshape(2 * e, 3 * w), src, dst).reshape(
            2, npad, 3 * w)
        for r in range(ncall)
    ]


def kernel(x, edge_index, edge_attr, u, params):
    src = edge_index[0]
    dst = edge_index[1]
    n, d = x.shape
    npad = _NPAD
    xp = jnp.pad(x, ((0, npad - n), (0, 0)))
    xq = xp.reshape(npad, 4, 32).transpose(1, 0, 2)
    xhs1 = [xq[2 * r:2 * r + 2].reshape(2 * npad, 32) for r in range(2)]
    aggs1 = _gine_aggs(xhs1, src, dst, edge_attr, params["em1"], params["c1"])
    xq1 = _node_layer(x, aggs1, params["c1"], params["lin1"], last=False)
    aggs2 = _gine_aggs([xq1.reshape(2 * npad, 32)], src, dst, edge_attr,
                       params["em2"], params["c2"])
    sum2 = _node_layer(xq1, aggs2, params["c2"], params["lin2"], last=True)
    pooled = sum2 / n
    return _apply_lin(params["fc"], jnp.concatenate([pooled, u], axis=1))
